# baseline (speedup 1.0000x reference)
import functools

import jax
import jax.numpy as jnp
from jax import lax
from jax.experimental import pallas as pl
from jax.experimental.pallas import tpu as pltpu
from jax.experimental.pallas import tpu_sc as plsc

N_FIELDS = 26
VOCAB_P1 = 100001
EMBED = 32
BATCH = 16384

NC = 2
NS = 16
NW = NC * NS
BPW = BATCH // NW
CHUNK = 128
NCHUNK = BPW // CHUNK


def _body(x3_hbm, tables_hbm, *rest):
    outs = rest[:N_FIELDS]
    idx_v, rows_v, gsem = rest[N_FIELDS:]
    wid = lax.axis_index("s") * NC + lax.axis_index("c")
    base = wid * BPW
    for i in range(N_FIELDS):
        pltpu.sync_copy(x3_hbm.at[i, pl.ds(wid * NCHUNK, NCHUNK)], idx_v)
        descs = []
        for c in range(NCHUNK):
            descs.append(
                pltpu.async_copy(
                    tables_hbm.at[i].at[idx_v.at[c]],
                    rows_v.at[pl.ds(c * CHUNK, CHUNK)],
                    gsem,
                )
            )
        for d in descs:
            d.wait()
        pltpu.sync_copy(rows_v, outs[i].at[pl.ds(base, BPW)])


@jax.jit
def _embed_all(x3, tables):
    mesh = plsc.VectorSubcoreMesh(core_axis_name="c", subcore_axis_name="s")
    f = pl.kernel(
        _body,
        mesh=mesh,
        out_type=[jax.ShapeDtypeStruct((BATCH, EMBED), jnp.float32)] * N_FIELDS,
        scratch_types=[
            pltpu.VMEM((NCHUNK, CHUNK), jnp.int32),
            pltpu.VMEM((BPW, EMBED), jnp.float32),
            pltpu.SemaphoreType.DMA,
        ],
        compiler_params=pltpu.CompilerParams(use_tc_tiling_on_sc=False),
    )
    return f(x3, tables)


def kernel(x, tables):
    x3 = x.T.reshape(N_FIELDS, BATCH // CHUNK, CHUNK)
    return tuple(_embed_all(x3, tables))


# speedup vs baseline: 2.4274x; 2.4274x over previous
import functools

import jax
import jax.numpy as jnp
from jax import lax
from jax.experimental import pallas as pl
from jax.experimental.pallas import tpu as pltpu
from jax.experimental.pallas import tpu_sc as plsc

N_FIELDS = 26
VOCAB_P1 = 100001
EMBED = 32
BATCH = 16384

BC = 1024
NBJ = 98
VP = NBJ * BC
FLAT = N_FIELDS * VP * EMBED

NC = 2
NS = 16
NW = NC * NS
BPW = BATCH // NW
CHUNK = 128
NCHUNK = BPW // CHUNK


def _tr_body(tt_ref, o_ref):
    t = tt_ref[0]
    tt = t.T.reshape(BC // 4, 4, EMBED)
    o_ref[...] = jnp.concatenate([tt[:, k, :] for k in range(4)], axis=1)


def _sc_body(x3_hbm, tflat_hbm, *rest):
    outs = rest[:N_FIELDS]
    idx_v, rows_v, gsem = rest[N_FIELDS:]
    wid = lax.axis_index("s") * NC + lax.axis_index("c")
    base = wid * BPW
    for i in range(N_FIELDS):
        pltpu.sync_copy(x3_hbm.at[i, pl.ds(wid * NCHUNK, NCHUNK)], idx_v)
        descs = []
        for c in range(NCHUNK):
            descs.append(
                pltpu.async_copy(
                    tflat_hbm.at[pl.ds(i * VP, VP)].at[idx_v.at[c]],
                    rows_v.at[pl.ds(c * CHUNK, CHUNK)],
                    gsem,
                )
            )
        for d in descs:
            d.wait()
        pltpu.sync_copy(rows_v, outs[i].at[pl.ds(base, BPW)])


@jax.jit
def _embed_all(x3, tablesT):
    zflat = pl.pallas_call(
        _tr_body,
        grid=(N_FIELDS, NBJ),
        in_specs=[pl.BlockSpec((1, EMBED, BC), lambda f, j: (f, 0, j))],
        out_specs=pl.BlockSpec(
            (BC * EMBED // 128, 128), lambda f, j: (f * NBJ + j, 0)
        ),
        out_shape=jax.ShapeDtypeStruct((FLAT // 128, 128), jnp.float32),
    )(tablesT)
    z2d = zflat.reshape(N_FIELDS * VP, EMBED)

    mesh = plsc.VectorSubcoreMesh(core_axis_name="c", subcore_axis_name="s")
    f = pl.kernel(
        _sc_body,
        mesh=mesh,
        out_type=[jax.ShapeDtypeStruct((BATCH, EMBED), jnp.float32)] * N_FIELDS,
        scratch_types=[
            pltpu.VMEM((NCHUNK, CHUNK), jnp.int32),
            pltpu.VMEM((BPW, EMBED), jnp.float32),
            pltpu.SemaphoreType.DMA,
        ],
        compiler_params=pltpu.CompilerParams(use_tc_tiling_on_sc=False),
    )
    return f(x3, z2d)


def kernel(x, tables):
    x3 = x.T.reshape(N_FIELDS, BATCH // CHUNK, CHUNK)
    tablesT = jnp.transpose(tables, (0, 2, 1))
    return tuple(_embed_all(x3, tablesT))
